# baseline (speedup 1.0000x reference)
import functools

import jax
import jax.numpy as jnp
from jax import lax
from jax.experimental import pallas as pl
from jax.experimental.pallas import tpu as pltpu
from jax.experimental.pallas import tpu_sc as plsc

NW = 32
LANES = 16
G = 256

_i32 = jnp.int32
_f32 = jnp.float32

_MESH = functools.partial(
    plsc.VectorSubcoreMesh, core_axis_name="c", subcore_axis_name="s",
    num_cores=2, num_subcores=16)

_CP = pltpu.CompilerParams(needs_layout_passes=False,
                           use_tc_tiling_on_sc=False)


def _wid():
    return lax.axis_index("s") * 2 + lax.axis_index("c")


def _iota():
    return lax.broadcasted_iota(_i32, (LANES,), 0)


def _splat(s):
    return jnp.zeros((LANES,), _i32) + s


def _rsqrt16(v):
    i = plsc.bitcast(v, _i32)
    y = plsc.bitcast(jnp.full((LANES,), 0x5F3759DF, _i32) - (i >> 1), _f32)
    for _ in range(3):
        y = y * (1.5 - 0.5 * v * y * y)
    return y


def _make_kernels(N, E, H, L):
    NPT = N // NW
    NPTP = -(-NPT // LANES) * LANES
    CH = 2048 if E >= NW * 4096 else max(16, min(1024, (E // NW) // 8 * 8))
    slack = max(1024, int(15 * (E // NW) ** 0.5))
    CAP = min(E + CH, E // NW + slack)
    CAP = min(-(-CAP // CH) * CH, -(-E // CH) * CH)
    CH0 = 16
    for c in range(4096, 15, -16):
        if E % c == 0:
            CH0 = c
            break
    H2 = 2 * H

    def _lanes():
        iv = _iota()
        m8 = iv < H
        hi8 = jnp.logical_not(m8)
        f8 = iv % H
        init_acc = jnp.where(m8, jnp.full((LANES,), -jnp.inf, _f32),
                             jnp.zeros((LANES,), _f32))
        return iv, m8, hi8, f8, init_acc

    @functools.partial(
        pl.kernel,
        out_type=(jax.ShapeDtypeStruct((NW, CAP), _i32),
                  jax.ShapeDtypeStruct((NW, CAP), _i32),
                  jax.ShapeDtypeStruct((NW, LANES), _i32),
                  jax.ShapeDtypeStruct((NW, NPTP), _i32)),
        mesh=_MESH(), compiler_params=_CP,
        scratch_types=[pltpu.VMEM((CH0,), _i32), pltpu.VMEM((CH0,), _i32),
                       pltpu.VMEM((CAP,), _i32), pltpu.VMEM((CAP,), _i32),
                       pltpu.VMEM((NPTP,), _i32), pltpu.VMEM((NPTP,), _i32),
                       pltpu.VMEM((LANES,), _i32)])
    def k0a(dst_h, src_h, eld_o, els_o, cnt_o, rs_o,
            dbuf, sbuf, eldv, elsv, degv, rsv, cbuf):
        iv, m8, hi8, f8, init_acc = _lanes()
        w = _wid()
        s_ax = lax.axis_index("s")
        sbase = s_ax * NPT
        lo = w * NPT
        hi = lo + NPT
        z16 = jnp.zeros((LANES,), _i32)
        one16 = jnp.full((LANES,), 1, _i32)

        def init_e(i, _):
            eldv[pl.ds(i * LANES, LANES)] = z16
            elsv[pl.ds(i * LANES, LANES)] = z16
            return 0
        lax.fori_loop(0, CAP // LANES, init_e, 0)

        def init_d(i, _):
            degv[pl.ds(i * LANES, LANES)] = z16
            return 0
        lax.fori_loop(0, NPTP // LANES, init_d, 0)

        def chunk(c, cntv):
            c0 = c * CH0
            pltpu.sync_copy(dst_h.at[pl.ds(c0, CH0)], dbuf)
            pltpu.sync_copy(src_h.at[pl.ds(c0, CH0)], sbuf)

            def scan16(i, cv):
                d = dbuf[pl.ds(i * LANES, LANES)]
                sv = sbuf[pl.ds(i * LANES, LANES)]
                m = jnp.logical_and(d >= lo, d < hi)
                mi = m.astype(_i32)
                cums = jnp.cumsum(mi)
                pos = cv + cums - mi
                dl = d - lo
                dlc = jnp.clip(dl, 0, NPT - 1)
                plsc.store_scatter(eldv, [pos], dl, mask=m)
                plsc.store_scatter(elsv, [pos], sv, mask=m)
                plsc.addupdate_scatter(degv, [dlc], one16, mask=m)
                return cv + plsc.all_reduce_population_count(m)
            return lax.fori_loop(0, CH0 // LANES, scan16, cntv)
        cntv = lax.fori_loop(0, E // CH0, chunk, jnp.zeros((LANES,), _i32))

        def prefix(i, carry):
            sl = pl.ds(i * LANES, LANES)
            d16 = degv[sl]
            c = jnp.cumsum(d16)
            rsv[sl] = carry + c - d16
            return carry + c[LANES - 1]
        lax.fori_loop(0, NPTP // LANES, prefix, jnp.int32(0))

        cbuf[...] = cntv
        pltpu.sync_copy(eldv, eld_o.at[w])
        pltpu.sync_copy(elsv, els_o.at[w])
        pltpu.sync_copy(cbuf, cnt_o.at[w])
        pltpu.sync_copy(rsv, rs_o.at[w])

    @functools.partial(
        pl.kernel,
        out_type=(jax.ShapeDtypeStruct((NW, CAP), _i32),
                  jax.ShapeDtypeStruct((NW, CAP), _i32)),
        mesh=_MESH(), compiler_params=_CP,
        scratch_types=[pltpu.VMEM((CH,), _i32), pltpu.VMEM((CH,), _i32),
                       pltpu.VMEM((CAP,), _i32), pltpu.VMEM((CAP,), _i32),
                       pltpu.VMEM((NPTP,), _i32), pltpu.VMEM((LANES,), _i32)])
    def k0c(eld_h, els_h, cnt_h, rs_h, elds_o, elss_o,
            dbuf, sbuf, eldv, elsv, wpv, cbuf):
        w = _wid()
        pltpu.sync_copy(cnt_h.at[w], cbuf)
        pltpu.sync_copy(rs_h.at[w], wpv)
        n_w = cbuf[...][0]
        z16 = jnp.zeros((LANES,), _i32)

        def init_e(i, _):
            eldv[pl.ds(i * LANES, LANES)] = z16
            elsv[pl.ds(i * LANES, LANES)] = z16
            return 0
        lax.fori_loop(0, CAP // LANES, init_e, 0)

        def chunk(c, _):
            c0 = c * CH
            cl = jnp.minimum(CH, n_w - c0)
            pltpu.sync_copy(eld_h.at[w, pl.ds(c0, CH)], dbuf)
            pltpu.sync_copy(els_h.at[w, pl.ds(c0, CH)], sbuf)

            def place(e, _):
                ev = _splat(e)
                dv = plsc.load_gather(dbuf, [ev])
                sv = plsc.load_gather(sbuf, [ev])
                pv = plsc.load_gather(wpv, [dv])
                plsc.store_scatter(wpv, [dv], pv + 1)
                plsc.store_scatter(eldv, [pv], dv)
                plsc.store_scatter(elsv, [pv], sv)
                return 0
            lax.fori_loop(0, cl, place, 0)
            return 0
        lax.fori_loop(0, (n_w + CH - 1) // CH, chunk, 0)
        pltpu.sync_copy(eldv, elds_o.at[w])
        pltpu.sync_copy(elsv, elss_o.at[w])

    @functools.partial(
        pl.kernel,
        out_type=(jax.ShapeDtypeStruct((NW, G, LANES), _f32),
                  jax.ShapeDtypeStruct((NW, G), _f32),
                  jax.ShapeDtypeStruct((NW, G), _i32)),
        mesh=_MESH(), compiler_params=_CP,
        scratch_types=[pltpu.VMEM((NPTP, H), _f32), pltpu.VMEM((NPTP,), _i32),
                       pltpu.VMEM((G, LANES), _f32), pltpu.VMEM((G,), _f32),
                       pltpu.VMEM((G,), _i32)])
    def k0b(x_h, batch_h, pool_o, cnt_o, grs_o, xv, bv, pacc, cacc, grsv):
        iv, m8, hi8, f8, init_acc = _lanes()
        w = _wid()
        lo = w * NPT
        pltpu.sync_copy(x_h.at[pl.ds(lo, NPT)], xv.at[pl.ds(0, NPT)])
        pltpu.sync_copy(batch_h.at[w], bv)

        def initp(g, _):
            plsc.store_scatter(pacc, [_splat(g), iv], init_acc)
            return 0
        lax.fori_loop(0, G, initp, 0)

        def initc(i, _):
            cacc[pl.ds(i * LANES, LANES)] = jnp.zeros((LANES,), _f32)
            return 0
        lax.fori_loop(0, G // LANES, initc, 0)

        def node(n, _):
            nv = _splat(n)
            gv = plsc.load_gather(bv, [nv])
            y = plsc.load_gather(xv, [nv, f8])
            p = plsc.load_gather(pacc, [gv, iv])
            plsc.store_scatter(pacc, [gv, iv],
                               jnp.where(m8, jnp.maximum(p, y), p + y))
            c = plsc.load_gather(cacc, [gv])
            plsc.store_scatter(cacc, [gv], c + 1.0)
            return 0
        lax.fori_loop(0, NPT, node, 0)

        def gprefix(i, carry):
            sl = pl.ds(i * LANES, LANES)
            d16 = cacc[sl].astype(_i32)
            c = jnp.cumsum(d16)
            grsv[sl] = carry + c - d16
            return carry + c[LANES - 1]
        lax.fori_loop(0, G // LANES, gprefix, jnp.int32(0))
        pltpu.sync_copy(pacc, pool_o.at[w])
        pltpu.sync_copy(cacc, cnt_o.at[w])
        pltpu.sync_copy(grsv, grs_o.at[w])

    @functools.partial(
        pl.kernel,
        out_type=(jax.ShapeDtypeStruct((N, H), _f32),
                  jax.ShapeDtypeStruct((NW, G, LANES), _f32)),
        mesh=_MESH(), compiler_params=_CP,
        scratch_types=[pltpu.VMEM((CH,), _i32), pltpu.VMEM((CH,), _i32),
                       pltpu.VMEM((CH, H), _f32),
                       pltpu.VMEM((NPTP * H,), _f32),
                       pltpu.VMEM((NPTP * H,), _f32),
                       pltpu.VMEM((NPTP, H), _f32),
                       pltpu.VMEM((NPTP,), _f32), pltpu.VMEM((NPTP,), _i32),
                       pltpu.VMEM((G,), _i32),
                       pltpu.VMEM((G, LANES), _f32),
                       pltpu.VMEM((H2 * H + LANES,), _f32),
                       pltpu.VMEM((H * H + LANES,), _f32),
                       pltpu.VMEM((LANES,), _i32)])
    def klayer(x_h, eld_h, els_h, cnt_h, rs_h, grs_h, wl_h, wr_h,
               xn_h, pool_o,
               dbuf, sbuf, msgv, mxv, smv, xbuf, rdv, rsv, bv, pacc,
               wlv, wrv, cntv_b):
        iv, m8, hi8, f8, init_acc = _lanes()
        w = _wid()
        lo = w * NPT
        pltpu.sync_copy(cnt_h.at[w], cntv_b)
        pltpu.sync_copy(wl_h, wlv)
        pltpu.sync_copy(wr_h, wrv)
        pltpu.sync_copy(x_h.at[pl.ds(lo, NPT)], xbuf.at[pl.ds(0, NPT)])
        pltpu.sync_copy(rs_h.at[w], rsv)
        pltpu.sync_copy(grs_h.at[w], bv)
        n_w = cntv_b[...][0]
        wls = []
        for k in range(H2):
            row = wlv[pl.ds(k * H, LANES)]
            wls.append([row[j] for j in range(H)])
        wrs = []
        for i in range(H):
            row = wrv[pl.ds(i * H, LANES)]
            wrs.append([row[j] for j in range(H)])

        neg16 = jnp.full((LANES,), -jnp.inf, _f32)
        z16f = jnp.zeros((LANES,), _f32)
        rows2 = (iv >= H).astype(_i32)

        def inita(i, _):
            mxv[pl.ds(i * LANES, LANES)] = neg16
            smv[pl.ds(i * LANES, LANES)] = z16f
            return 0
        lax.fori_loop(0, NPTP * H // LANES, inita, 0)

        def rdeg(i, _):
            sl = pl.ds(i * LANES, LANES)
            a = rsv[sl]
            b = plsc.load_gather(rsv, [jnp.minimum(_splat(i * LANES) + iv + 1,
                                                   _splat(NPTP - 1))])
            bfix = jnp.where(_splat(i * LANES) + iv + 1 >= NPTP,
                             _splat(n_w), b)
            deg = (bfix - a).astype(_f32)
            rdv[sl] = 1.0 / jnp.maximum(deg, 1.0)
            return 0
        lax.fori_loop(0, NPTP // LANES, rdeg, 0)

        def chunk(c, _):
            c0 = c * CH
            cl = jnp.minimum(CH, n_w - c0)
            cend = c0 + cl
            pltpu.sync_copy(eld_h.at[w, pl.ds(c0, CH)], dbuf)
            pltpu.sync_copy(els_h.at[w, pl.ds(c0, CH)], sbuf)
            pltpu.sync_copy(x_h.at[sbuf], msgv)
            nlo = plsc.load_gather(dbuf, [_splat(0)])[0]
            nhi = plsc.load_gather(dbuf, [_splat(cl - 1)])[0]
            m0 = nlo // 2

            def pairs(mi_, _):
                m = m0 + mi_
                n0 = 2 * m
                g3 = plsc.load_gather(rsv, [_splat(n0) + jnp.minimum(iv, 2)])
                a0 = g3[0]
                a1 = g3[1]
                a2 = g3[2]
                kA0 = jnp.maximum(a0, c0)
                dA = jnp.maximum(jnp.minimum(a1, cend) - kA0, 0)
                kB0 = jnp.maximum(a1, c0)
                dB = jnp.maximum(jnp.minimum(a2, cend) - kB0, 0)
                kmax = jnp.maximum(dA, dB)
                base = jnp.where(m8, _splat(kA0), _splat(kB0)) - c0
                dcnt = jnp.where(m8, _splat(dA), _splat(dB))

                def acc_k(k, carry):
                    am, asum = carry
                    rowl = base + k
                    valid = (jnp.zeros((LANES,), _i32) + k) < dcnt
                    rc = jnp.minimum(rowl, CH - 1)
                    msg = plsc.load_gather(msgv, [rc, f8])
                    am = jnp.maximum(am, jnp.where(valid, msg, -jnp.inf))
                    asum = asum + jnp.where(valid, msg, 0.0)
                    return (am, asum)
                am, asum = lax.fori_loop(0, kmax, acc_k, (neg16, z16f))
                off = m * LANES
                sl = pl.ds(off, LANES)
                mxv[sl] = jnp.maximum(mxv[sl], am)
                smv[sl] = smv[sl] + asum
                return 0
            lax.fori_loop(0, nhi // 2 - m0 + 1, pairs, 0)
            return 0
        lax.fori_loop(0, (n_w + CH - 1) // CH, chunk, 0)

        def fin(i, _):
            sl = pl.ds(i * LANES, LANES)
            v = mxv[sl]
            mxv[sl] = jnp.where(v == -jnp.inf, 0.0, v)
            r2 = _splat(2 * i) + rows2
            rd = plsc.load_gather(rdv, [r2])
            smv[sl] = smv[sl] * rd
            return 0
        lax.fori_loop(0, NPTP * H // LANES, fin, 0)

        def blk(nb, _):
            nodes = _splat(nb * LANES) + iv
            nflat = nodes * H
            h = [jnp.zeros((LANES,), _f32) for _ in range(H)]
            for k in range(H):
                ak = plsc.load_gather(mxv, [nflat + k])
                for j in range(H):
                    h[j] = h[j] + ak * wls[k][j]
            for k in range(H):
                ak = plsc.load_gather(smv, [nflat + k])
                for j in range(H):
                    h[j] = h[j] + ak * wls[H + k][j]
            for i in range(H):
                xi = plsc.load_gather(xbuf, [nodes, _splat(i)])
                for j in range(H):
                    h[j] = h[j] + xi * wrs[i][j]
            for j in range(H):
                y = h[j] * (1.0 / (1.0 + jnp.exp(-h[j])))
                plsc.store_scatter(xbuf, [nodes, _splat(j)], y)
            return 0
        lax.fori_loop(0, NPTP // LANES, blk, 0)

        def gpool(g, _):
            gg = plsc.load_gather(bv, [jnp.minimum(_splat(g) + iv,
                                                   _splat(G - 1))])
            a = gg[0]
            b = jnp.where(g + 1 >= G, jnp.int32(NPT),
                          jnp.minimum(gg[1], NPT))
            bs = _splat(b)

            def acc2(k, carry):
                am, asum = carry
                r = _splat(a + 2 * k) + rows2
                rsw = _splat(a + 2 * k + 1) - rows2
                v1 = r < bs
                v2 = rsw < bs
                y = plsc.load_gather(
                    xbuf, [jnp.minimum(r, _splat(NPTP - 1)), f8])
                ysw = plsc.load_gather(
                    xbuf, [jnp.minimum(rsw, _splat(NPTP - 1)), f8])
                am = jnp.maximum(am, jnp.maximum(
                    jnp.where(v1, y, -jnp.inf), jnp.where(v2, ysw, -jnp.inf)))
                asum = asum + jnp.where(v1, y, 0.0) + jnp.where(v2, ysw, 0.0)
                return (am, asum)
            am, asum = lax.fori_loop(0, (b - a + 1) // 2, acc2,
                                     (neg16, z16f))
            plsc.store_scatter(pacc, [_splat(g), iv],
                               jnp.where(m8, am, asum))
            return 0
        lax.fori_loop(0, G, gpool, 0)

        pltpu.sync_copy(xbuf.at[pl.ds(0, NPT)], xn_h.at[pl.ds(lo, NPT)])
        pltpu.sync_copy(pacc, pool_o.at[w])

    NST = L + 1
    D1 = 24 * NST
    FW = -(-D1 // LANES) * LANES + 8
    DH = 4 * H
    NT9 = G // LANES

    @functools.partial(
        pl.kernel,
        out_type=jax.ShapeDtypeStruct((G * 2,), _f32),
        mesh=_MESH(), compiler_params=_CP,
        scratch_types=[pltpu.VMEM((LANES, LANES), _f32),
                       pltpu.VMEM((LANES,), _f32),
                       pltpu.VMEM((LANES, FW), _f32),
                       pltpu.VMEM((D1, DH), _f32),
                       pltpu.VMEM((DH,), _f32), pltpu.VMEM((DH,), _f32),
                       pltpu.VMEM((2 * DH,), _f32),
                       pltpu.VMEM((2 * LANES,), _f32)])
    def k9(cnt_h, w1_h, lng_h, lnb_h, w2_h, *rest):
        pools_h = rest[:NST]
        out_h = rest[NST]
        pbuf, rcv, fbuf, w1v, lgv, lbv, w2v, obuf = rest[NST + 1:]
        iv, m8, hi8, f8, init_acc = _lanes()
        w = _wid()

        @pl.when(w < NT9)
        def _():
            g0 = w * LANES
            pltpu.sync_copy(w1_h, w1v)
            pltpu.sync_copy(lng_h, lgv)
            pltpu.sync_copy(lnb_h, lbv)
            pltpu.sync_copy(w2_h, w2v)

            def csum(p, cv):
                pltpu.sync_copy(cnt_h.at[p, pl.ds(g0, LANES)], rcv)
                return cv + rcv[...]
            cntv = lax.fori_loop(0, NW, csum, jnp.zeros((LANES,), _f32))
            rcv[...] = 1.0 / jnp.maximum(cntv, 1.0)

            for st in range(NST):
                def comb(p, accs, st=st):
                    pltpu.sync_copy(pools_h[st].at[p, pl.ds(g0, LANES)], pbuf)
                    out = []
                    for q in range(LANES):
                        v = plsc.load_gather(pbuf, [_splat(q), iv])
                        out.append(jnp.where(m8, jnp.maximum(accs[q], v),
                                             accs[q] + v))
                    return tuple(out)
                accs = lax.fori_loop(0, NW, comb,
                                     tuple(init_acc for _ in range(LANES)))
                for q in range(LANES):
                    vq = jnp.where(accs[q] == -jnp.inf, 0.0, accs[q])
                    rcq = plsc.load_gather(rcv, [_splat(q)])
                    meanv = vq * rcq
                    qv = _splat(q)
                    cmean = jnp.maximum(st * 24 + iv - H, 0)
                    plsc.store_scatter(fbuf, [qv, cmean], meanv, mask=hi8)
                    plsc.store_scatter(fbuf, [qv, st * 24 + H + iv], vq,
                                       mask=m8)
                    plsc.store_scatter(fbuf, [qv, st * 24 + H + iv], vq,
                                       mask=hi8)

            def graph(g, ovs):
                gv = _splat(g)

                def dot(k, hh):
                    kv = _splat(k)
                    fgk = plsc.load_gather(fbuf, [gv, kv])
                    wa = plsc.load_gather(w1v, [kv, iv])
                    wb = plsc.load_gather(w1v, [kv, iv + LANES])
                    return (hh[0] + fgk * wa, hh[1] + fgk * wb)
                ha, hb = lax.fori_loop(0, D1, dot,
                                       (jnp.zeros((LANES,), _f32),
                                        jnp.zeros((LANES,), _f32)))
                ha = ha * (1.0 / (1.0 + jnp.exp(-ha)))
                hb = hb * (1.0 / (1.0 + jnp.exp(-hb)))
                mu = (jnp.sum(ha) + jnp.sum(hb)) * (1.0 / DH)
                var = (jnp.sum(ha * ha) + jnp.sum(hb * hb)) * (1.0 / DH) - mu * mu
                rstd = _rsqrt16(jnp.zeros((LANES,), _f32) + var + 1e-5)
                na = (ha - mu) * rstd * lgv[pl.ds(0, LANES)] \
                    + lbv[pl.ds(0, LANES)]
                nb_ = (hb - mu) * rstd * lgv[pl.ds(LANES, LANES)] \
                    + lbv[pl.ds(LANES, LANES)]
                w20a = w2v[pl.ds(0, LANES)]
                w20b = w2v[pl.ds(LANES, LANES)]
                w21a = w2v[pl.ds(2 * LANES, LANES)]
                w21b = w2v[pl.ds(3 * LANES, LANES)]
                o0 = jnp.sum(na * w20a) + jnp.sum(nb_ * w20b)
                o1 = jnp.sum(na * w21a) + jnp.sum(nb_ * w21b)
                t = 2 * g
                oa = jnp.where(iv == t, o0, ovs[0])
                oa = jnp.where(iv == t + 1, o1, oa)
                ob = jnp.where(iv == t - LANES, o0, ovs[1])
                ob = jnp.where(iv == t - LANES + 1, o1, ob)
                return (oa, ob)
            oa, ob = lax.fori_loop(0, LANES, graph,
                                   (jnp.zeros((LANES,), _f32),
                                    jnp.zeros((LANES,), _f32)))
            obuf[pl.ds(0, LANES)] = oa
            obuf[pl.ds(LANES, LANES)] = ob
            pltpu.sync_copy(obuf, out_h.at[pl.ds(w * 2 * LANES, 2 * LANES)])

    return k0a, k0c, k0b, klayer, k9, NPT, NPTP


def kernel(x, edge_index, batch, Wl0, Wr0, Wl_rest, Wr_rest,
           ro_W1, ln_g, ln_b, ro_W2):
    N, NIN = x.shape
    E = edge_index.shape[1]
    H = Wr0.shape[1]
    L = Wl_rest.shape[0] + 1
    k0a, k0c, k0b, klayer, k9, NPT, NPTP = _make_kernels(N, E, H, L)

    xp = jnp.pad(x.astype(_f32), ((0, 0), (0, H - NIN)))
    Wl0p = jnp.zeros((2 * H, H), _f32)
    Wl0p = Wl0p.at[0:NIN].set(Wl0[0:NIN]).at[H:H + NIN].set(Wl0[NIN:2 * NIN])
    Wr0p = jnp.zeros((H, H), _f32).at[0:NIN].set(Wr0)
    Wls = jnp.concatenate([Wl0p[None], Wl_rest], axis=0)
    Wrs = jnp.concatenate([Wr0p[None], Wr_rest], axis=0)
    Wls = jnp.pad(Wls.reshape(L, 2 * H * H), ((0, 0), (0, LANES)))
    Wrs = jnp.pad(Wrs.reshape(L, H * H), ((0, 0), (0, LANES)))

    D1 = 24 * (L + 1)
    DH = 4 * H
    W1p = jnp.zeros((D1, DH), _f32)
    W1p = W1p.at[0:NIN].set(ro_W1[0:NIN])
    W1p = W1p.at[H:H + NIN].set(ro_W1[NIN:2 * NIN])
    W1p = W1p.at[2 * H:2 * H + NIN].set(ro_W1[2 * NIN:3 * NIN])
    for s in range(1, L + 1):
        rb = 3 * NIN + (s - 1) * 3 * H
        W1p = W1p.at[24 * s:24 * s + 3 * H].set(ro_W1[rb:rb + 3 * H])
    W2t = ro_W2.T.reshape(-1)

    src = edge_index[0]
    dst = edge_index[1]
    batch2 = jnp.pad(batch.reshape(NW, NPT), ((0, 0), (0, NPTP - NPT)))

    eldu, elsu, cnt, rs = k0a(dst, src)
    elds, elss = k0c(eldu, elsu, cnt, rs)
    pools0, cntp, grs = k0b(xp, batch2)
    pools = [pools0]
    xc = xp
    for l in range(L):
        xc, pp = klayer(xc, elds, elss, cnt, rs, grs, Wls[l], Wrs[l])
        pools.append(pp)
    out = k9(cntp, W1p, ln_g, ln_b, W2t, *pools)
    return out.reshape(G, 2)


# speedup vs baseline: 1.0882x; 1.0882x over previous
import functools

import jax
import jax.numpy as jnp
from jax import lax
from jax.experimental import pallas as pl
from jax.experimental.pallas import tpu as pltpu
from jax.experimental.pallas import tpu_sc as plsc

NW = 32
LANES = 16
G = 256

_i32 = jnp.int32
_f32 = jnp.float32

_MESH = functools.partial(
    plsc.VectorSubcoreMesh, core_axis_name="c", subcore_axis_name="s",
    num_cores=2, num_subcores=16)

_CP = pltpu.CompilerParams(needs_layout_passes=False,
                           use_tc_tiling_on_sc=False)


def _wid():
    return lax.axis_index("s") * 2 + lax.axis_index("c")


def _iota():
    return lax.broadcasted_iota(_i32, (LANES,), 0)


def _splat(s):
    return jnp.zeros((LANES,), _i32) + s


def _rsqrt16(v):
    i = plsc.bitcast(v, _i32)
    y = plsc.bitcast(jnp.full((LANES,), 0x5F3759DF, _i32) - (i >> 1), _f32)
    for _ in range(3):
        y = y * (1.5 - 0.5 * v * y * y)
    return y


def _make_kernels(N, E, H, L):
    NPT = N // NW
    NPTP = -(-NPT // LANES) * LANES
    CH = 1024 if E >= NW * 2048 else max(16, min(1024, (E // NW) // 8 * 8))
    slack = max(1024, int(15 * (E // NW) ** 0.5))
    CAP = min(E + CH, E // NW + slack)
    CAP = min(-(-CAP // CH) * CH, -(-E // CH) * CH)
    CH0 = 16
    for c in range(4096, 15, -16):
        if E % c == 0:
            CH0 = c
            break
    H2 = 2 * H

    def _lanes():
        iv = _iota()
        m8 = iv < H
        hi8 = jnp.logical_not(m8)
        f8 = iv % H
        init_acc = jnp.where(m8, jnp.full((LANES,), -jnp.inf, _f32),
                             jnp.zeros((LANES,), _f32))
        return iv, m8, hi8, f8, init_acc

    @functools.partial(
        pl.kernel,
        out_type=(jax.ShapeDtypeStruct((NW, CAP), _i32),
                  jax.ShapeDtypeStruct((NW, CAP), _i32),
                  jax.ShapeDtypeStruct((NW, LANES), _i32),
                  jax.ShapeDtypeStruct((NW, NPTP), _i32)),
        mesh=_MESH(), compiler_params=_CP,
        scratch_types=[pltpu.VMEM((CH0,), _i32), pltpu.VMEM((CH0,), _i32),
                       pltpu.VMEM((CAP,), _i32), pltpu.VMEM((CAP,), _i32),
                       pltpu.VMEM((NPTP,), _i32), pltpu.VMEM((NPTP,), _i32),
                       pltpu.VMEM((LANES,), _i32)])
    def k0a(dst_h, src_h, eld_o, els_o, cnt_o, rs_o,
            dbuf, sbuf, eldv, elsv, degv, rsv, cbuf):
        iv, m8, hi8, f8, init_acc = _lanes()
        w = _wid()
        s_ax = lax.axis_index("s")
        sbase = s_ax * NPT
        lo = w * NPT
        hi = lo + NPT
        z16 = jnp.zeros((LANES,), _i32)
        one16 = jnp.full((LANES,), 1, _i32)

        def init_e(i, _):
            eldv[pl.ds(i * LANES, LANES)] = z16
            elsv[pl.ds(i * LANES, LANES)] = z16
            return 0
        lax.fori_loop(0, CAP // LANES, init_e, 0)

        def init_d(i, _):
            degv[pl.ds(i * LANES, LANES)] = z16
            return 0
        lax.fori_loop(0, NPTP // LANES, init_d, 0)

        def chunk(c, cntv):
            c0 = c * CH0
            pltpu.sync_copy(dst_h.at[pl.ds(c0, CH0)], dbuf)
            pltpu.sync_copy(src_h.at[pl.ds(c0, CH0)], sbuf)

            def scan16(i, cv):
                d = dbuf[pl.ds(i * LANES, LANES)]
                sv = sbuf[pl.ds(i * LANES, LANES)]
                m = jnp.logical_and(d >= lo, d < hi)
                mi = m.astype(_i32)
                cums = jnp.cumsum(mi)
                pos = cv + cums - mi
                dl = d - lo
                dlc = jnp.clip(dl, 0, NPT - 1)
                plsc.store_scatter(eldv, [pos], dl, mask=m)
                plsc.store_scatter(elsv, [pos], sv, mask=m)
                plsc.addupdate_scatter(degv, [dlc], one16, mask=m)
                return cv + plsc.all_reduce_population_count(m)
            return lax.fori_loop(0, CH0 // LANES, scan16, cntv)
        cntv = lax.fori_loop(0, E // CH0, chunk, jnp.zeros((LANES,), _i32))

        def prefix(i, carry):
            sl = pl.ds(i * LANES, LANES)
            d16 = degv[sl]
            c = jnp.cumsum(d16)
            rsv[sl] = carry + c - d16
            return carry + c[LANES - 1]
        lax.fori_loop(0, NPTP // LANES, prefix, jnp.int32(0))

        cbuf[...] = cntv
        pltpu.sync_copy(eldv, eld_o.at[w])
        pltpu.sync_copy(elsv, els_o.at[w])
        pltpu.sync_copy(cbuf, cnt_o.at[w])
        pltpu.sync_copy(rsv, rs_o.at[w])

    @functools.partial(
        pl.kernel,
        out_type=(jax.ShapeDtypeStruct((NW, CAP), _i32),
                  jax.ShapeDtypeStruct((NW, CAP), _i32)),
        mesh=_MESH(), compiler_params=_CP,
        scratch_types=[pltpu.VMEM((CH,), _i32), pltpu.VMEM((CH,), _i32),
                       pltpu.VMEM((CAP,), _i32), pltpu.VMEM((CAP,), _i32),
                       pltpu.VMEM((NPTP,), _i32), pltpu.VMEM((LANES,), _i32)])
    def k0c(eld_h, els_h, cnt_h, rs_h, elds_o, elss_o,
            dbuf, sbuf, eldv, elsv, wpv, cbuf):
        w = _wid()
        pltpu.sync_copy(cnt_h.at[w], cbuf)
        pltpu.sync_copy(rs_h.at[w], wpv)
        n_w = cbuf[...][0]
        z16 = jnp.zeros((LANES,), _i32)

        def init_e(i, _):
            eldv[pl.ds(i * LANES, LANES)] = z16
            elsv[pl.ds(i * LANES, LANES)] = z16
            return 0
        lax.fori_loop(0, CAP // LANES, init_e, 0)

        def chunk(c, _):
            c0 = c * CH
            cl = jnp.minimum(CH, n_w - c0)
            pltpu.sync_copy(eld_h.at[w, pl.ds(c0, CH)], dbuf)
            pltpu.sync_copy(els_h.at[w, pl.ds(c0, CH)], sbuf)

            def place(e, _):
                ev = _splat(e)
                dv = plsc.load_gather(dbuf, [ev])
                sv = plsc.load_gather(sbuf, [ev])
                pv = plsc.load_gather(wpv, [dv])
                plsc.store_scatter(wpv, [dv], pv + 1)
                plsc.store_scatter(eldv, [pv], dv)
                plsc.store_scatter(elsv, [pv], sv)
                return 0
            lax.fori_loop(0, cl, place, 0)
            return 0
        lax.fori_loop(0, (n_w + CH - 1) // CH, chunk, 0)
        pltpu.sync_copy(eldv, elds_o.at[w])
        pltpu.sync_copy(elsv, elss_o.at[w])

    @functools.partial(
        pl.kernel,
        out_type=(jax.ShapeDtypeStruct((NW, G, LANES), _f32),
                  jax.ShapeDtypeStruct((NW, G), _f32),
                  jax.ShapeDtypeStruct((NW, G), _i32)),
        mesh=_MESH(), compiler_params=_CP,
        scratch_types=[pltpu.VMEM((NPTP, H), _f32), pltpu.VMEM((NPTP,), _i32),
                       pltpu.VMEM((G, LANES), _f32), pltpu.VMEM((G,), _f32),
                       pltpu.VMEM((G,), _i32)])
    def k0b(x_h, batch_h, pool_o, cnt_o, grs_o, xv, bv, pacc, cacc, grsv):
        iv, m8, hi8, f8, init_acc = _lanes()
        w = _wid()
        lo = w * NPT
        pltpu.sync_copy(x_h.at[pl.ds(lo, NPT)], xv.at[pl.ds(0, NPT)])
        pltpu.sync_copy(batch_h.at[w], bv)

        def initp(g, _):
            plsc.store_scatter(pacc, [_splat(g), iv], init_acc)
            return 0
        lax.fori_loop(0, G, initp, 0)

        def initc(i, _):
            cacc[pl.ds(i * LANES, LANES)] = jnp.zeros((LANES,), _f32)
            return 0
        lax.fori_loop(0, G // LANES, initc, 0)

        def node(n, _):
            nv = _splat(n)
            gv = plsc.load_gather(bv, [nv])
            y = plsc.load_gather(xv, [nv, f8])
            p = plsc.load_gather(pacc, [gv, iv])
            plsc.store_scatter(pacc, [gv, iv],
                               jnp.where(m8, jnp.maximum(p, y), p + y))
            c = plsc.load_gather(cacc, [gv])
            plsc.store_scatter(cacc, [gv], c + 1.0)
            return 0
        lax.fori_loop(0, NPT, node, 0)

        def gprefix(i, carry):
            sl = pl.ds(i * LANES, LANES)
            d16 = cacc[sl].astype(_i32)
            c = jnp.cumsum(d16)
            grsv[sl] = carry + c - d16
            return carry + c[LANES - 1]
        lax.fori_loop(0, G // LANES, gprefix, jnp.int32(0))
        pltpu.sync_copy(pacc, pool_o.at[w])
        pltpu.sync_copy(cacc, cnt_o.at[w])
        pltpu.sync_copy(grsv, grs_o.at[w])

    @functools.partial(
        pl.kernel,
        out_type=(jax.ShapeDtypeStruct((N, H), _f32),
                  jax.ShapeDtypeStruct((NW, G, LANES), _f32)),
        mesh=_MESH(), compiler_params=_CP,
        scratch_types=[pltpu.VMEM((CH,), _i32), pltpu.VMEM((CH,), _i32),
                       pltpu.VMEM((CH, H), _f32),
                       pltpu.VMEM((NPTP * H,), _f32),
                       pltpu.VMEM((NPTP * H,), _f32),
                       pltpu.VMEM((NPTP, H), _f32),
                       pltpu.VMEM((NPTP,), _f32), pltpu.VMEM((NPTP,), _i32),
                       pltpu.VMEM((G,), _i32),
                       pltpu.VMEM((G, LANES), _f32),
                       pltpu.VMEM((H2 * H + LANES,), _f32),
                       pltpu.VMEM((H * H + LANES,), _f32),
                       pltpu.VMEM((LANES,), _i32)])
    def klayer(x_h, eld_h, els_h, cnt_h, rs_h, grs_h, wl_h, wr_h,
               xn_h, pool_o,
               dbuf, sbuf, msgv, mxv, smv, xbuf, rdv, rsv, bv, pacc,
               wlv, wrv, cntv_b):
        iv, m8, hi8, f8, init_acc = _lanes()
        w = _wid()
        lo = w * NPT
        pltpu.sync_copy(cnt_h.at[w], cntv_b)
        pltpu.sync_copy(wl_h, wlv)
        pltpu.sync_copy(wr_h, wrv)
        pltpu.sync_copy(x_h.at[pl.ds(lo, NPT)], xbuf.at[pl.ds(0, NPT)])
        pltpu.sync_copy(rs_h.at[w], rsv)
        pltpu.sync_copy(grs_h.at[w], bv)
        n_w = cntv_b[...][0]
        wls = []
        for k in range(H2):
            row = wlv[pl.ds(k * H, LANES)]
            wls.append([row[j] for j in range(H)])
        wrs = []
        for i in range(H):
            row = wrv[pl.ds(i * H, LANES)]
            wrs.append([row[j] for j in range(H)])

        neg16 = jnp.full((LANES,), -jnp.inf, _f32)
        z16f = jnp.zeros((LANES,), _f32)
        rows2 = (iv >= H).astype(_i32)

        def inita(i, _):
            mxv[pl.ds(i * LANES, LANES)] = neg16
            smv[pl.ds(i * LANES, LANES)] = z16f
            return 0
        lax.fori_loop(0, NPTP * H // LANES, inita, 0)

        def rdeg(i, _):
            sl = pl.ds(i * LANES, LANES)
            a = rsv[sl]
            b = plsc.load_gather(rsv, [jnp.minimum(_splat(i * LANES) + iv + 1,
                                                   _splat(NPTP - 1))])
            bfix = jnp.where(_splat(i * LANES) + iv + 1 >= NPTP,
                             _splat(n_w), b)
            deg = (bfix - a).astype(_f32)
            rdv[sl] = 1.0 / jnp.maximum(deg, 1.0)
            return 0
        lax.fori_loop(0, NPTP // LANES, rdeg, 0)

        def chunk(c, _):
            c0 = c * CH
            cl = jnp.minimum(CH, n_w - c0)
            cend = c0 + cl
            pltpu.sync_copy(eld_h.at[w, pl.ds(c0, CH)], dbuf)
            pltpu.sync_copy(els_h.at[w, pl.ds(c0, CH)], sbuf)
            pltpu.sync_copy(x_h.at[sbuf], msgv)
            nlo = plsc.load_gather(dbuf, [_splat(0)])[0]
            nhi = plsc.load_gather(dbuf, [_splat(cl - 1)])[0]
            m0 = nlo // 2

            def pairs(mi_, _):
                m = m0 + mi_
                n0 = 2 * m
                g3 = plsc.load_gather(rsv, [_splat(n0) + jnp.minimum(iv, 2)])
                a0 = g3[0]
                a1 = g3[1]
                a2 = g3[2]
                kA0 = jnp.maximum(a0, c0)
                dA = jnp.maximum(jnp.minimum(a1, cend) - kA0, 0)
                kB0 = jnp.maximum(a1, c0)
                dB = jnp.maximum(jnp.minimum(a2, cend) - kB0, 0)
                kmax = jnp.maximum(dA, dB)
                base = jnp.where(m8, _splat(kA0), _splat(kB0)) - c0
                dcnt = jnp.where(m8, _splat(dA), _splat(dB))

                def acc_k(k, carry):
                    am, asum = carry
                    rowl = base + k
                    valid = (jnp.zeros((LANES,), _i32) + k) < dcnt
                    rc = jnp.minimum(rowl, CH - 1)
                    msg = plsc.load_gather(msgv, [rc, f8])
                    am = jnp.maximum(am, jnp.where(valid, msg, -jnp.inf))
                    asum = asum + jnp.where(valid, msg, 0.0)
                    return (am, asum)
                am, asum = lax.fori_loop(0, kmax, acc_k, (neg16, z16f))
                off = m * LANES
                sl = pl.ds(off, LANES)
                mxv[sl] = jnp.maximum(mxv[sl], am)
                smv[sl] = smv[sl] + asum
                return 0
            lax.fori_loop(0, nhi // 2 - m0 + 1, pairs, 0)
            return 0
        lax.fori_loop(0, (n_w + CH - 1) // CH, chunk, 0)

        def fin(i, _):
            sl = pl.ds(i * LANES, LANES)
            v = mxv[sl]
            mxv[sl] = jnp.where(v == -jnp.inf, 0.0, v)
            r2 = _splat(2 * i) + rows2
            rd = plsc.load_gather(rdv, [r2])
            smv[sl] = smv[sl] * rd
            return 0
        lax.fori_loop(0, NPTP * H // LANES, fin, 0)

        def blk(nb, _):
            nodes = _splat(nb * LANES) + iv
            nflat = nodes * H
            h = [jnp.zeros((LANES,), _f32) for _ in range(H)]
            for k in range(H):
                ak = plsc.load_gather(mxv, [nflat + k])
                for j in range(H):
                    h[j] = h[j] + ak * wls[k][j]
            for k in range(H):
                ak = plsc.load_gather(smv, [nflat + k])
                for j in range(H):
                    h[j] = h[j] + ak * wls[H + k][j]
            for i in range(H):
                xi = plsc.load_gather(xbuf, [nodes, _splat(i)])
                for j in range(H):
                    h[j] = h[j] + xi * wrs[i][j]
            for j in range(H):
                y = h[j] * (1.0 / (1.0 + jnp.exp(-h[j])))
                plsc.store_scatter(xbuf, [nodes, _splat(j)], y)
            return 0
        lax.fori_loop(0, NPTP // LANES, blk, 0)

        def gpool(g, _):
            gg = plsc.load_gather(bv, [jnp.minimum(_splat(g) + iv,
                                                   _splat(G - 1))])
            a = gg[0]
            b = jnp.where(g + 1 >= G, jnp.int32(NPT),
                          jnp.minimum(gg[1], NPT))
            bs = _splat(b)

            def acc2(k, carry):
                am, asum = carry
                r = _splat(a + 2 * k) + rows2
                rsw = _splat(a + 2 * k + 1) - rows2
                v1 = r < bs
                v2 = rsw < bs
                y = plsc.load_gather(
                    xbuf, [jnp.minimum(r, _splat(NPTP - 1)), f8])
                ysw = plsc.load_gather(
                    xbuf, [jnp.minimum(rsw, _splat(NPTP - 1)), f8])
                am = jnp.maximum(am, jnp.maximum(
                    jnp.where(v1, y, -jnp.inf), jnp.where(v2, ysw, -jnp.inf)))
                asum = asum + jnp.where(v1, y, 0.0) + jnp.where(v2, ysw, 0.0)
                return (am, asum)
            am, asum = lax.fori_loop(0, (b - a + 1) // 2, acc2,
                                     (neg16, z16f))
            plsc.store_scatter(pacc, [_splat(g), iv],
                               jnp.where(m8, am, asum))
            return 0
        lax.fori_loop(0, G, gpool, 0)

        pltpu.sync_copy(xbuf.at[pl.ds(0, NPT)], xn_h.at[pl.ds(lo, NPT)])
        pltpu.sync_copy(pacc, pool_o.at[w])

    NST = L + 1
    D1 = 24 * NST
    FW = -(-D1 // LANES) * LANES + 8
    DH = 4 * H
    NT9 = G // LANES

    @functools.partial(
        pl.kernel,
        out_type=jax.ShapeDtypeStruct((G * 2,), _f32),
        mesh=_MESH(), compiler_params=_CP,
        scratch_types=[pltpu.VMEM((LANES, LANES), _f32),
                       pltpu.VMEM((LANES,), _f32),
                       pltpu.VMEM((LANES, FW), _f32),
                       pltpu.VMEM((D1, DH), _f32),
                       pltpu.VMEM((DH,), _f32), pltpu.VMEM((DH,), _f32),
                       pltpu.VMEM((2 * DH,), _f32),
                       pltpu.VMEM((2 * LANES,), _f32)])
    def k9(cnt_h, w1_h, lng_h, lnb_h, w2_h, *rest):
        pools_h = rest[:NST]
        out_h = rest[NST]
        pbuf, rcv, fbuf, w1v, lgv, lbv, w2v, obuf = rest[NST + 1:]
        iv, m8, hi8, f8, init_acc = _lanes()
        w = _wid()

        @pl.when(w < NT9)
        def _():
            g0 = w * LANES
            pltpu.sync_copy(w1_h, w1v)
            pltpu.sync_copy(lng_h, lgv)
            pltpu.sync_copy(lnb_h, lbv)
            pltpu.sync_copy(w2_h, w2v)

            def csum(p, cv):
                pltpu.sync_copy(cnt_h.at[p, pl.ds(g0, LANES)], rcv)
                return cv + rcv[...]
            cntv = lax.fori_loop(0, NW, csum, jnp.zeros((LANES,), _f32))
            rcv[...] = 1.0 / jnp.maximum(cntv, 1.0)

            for st in range(NST):
                def comb(p, accs, st=st):
                    pltpu.sync_copy(pools_h[st].at[p, pl.ds(g0, LANES)], pbuf)
                    out = []
                    for q in range(LANES):
                        v = plsc.load_gather(pbuf, [_splat(q), iv])
                        out.append(jnp.where(m8, jnp.maximum(accs[q], v),
                                             accs[q] + v))
                    return tuple(out)
                accs = lax.fori_loop(0, NW, comb,
                                     tuple(init_acc for _ in range(LANES)))
                for q in range(LANES):
                    vq = jnp.where(accs[q] == -jnp.inf, 0.0, accs[q])
                    rcq = plsc.load_gather(rcv, [_splat(q)])
                    meanv = vq * rcq
                    qv = _splat(q)
                    cmean = jnp.maximum(st * 24 + iv - H, 0)
                    plsc.store_scatter(fbuf, [qv, cmean], meanv, mask=hi8)
                    plsc.store_scatter(fbuf, [qv, st * 24 + H + iv], vq,
                                       mask=m8)
                    plsc.store_scatter(fbuf, [qv, st * 24 + H + iv], vq,
                                       mask=hi8)

            def graph(g, ovs):
                gv = _splat(g)

                def dot(k, hh):
                    kv = _splat(k)
                    fgk = plsc.load_gather(fbuf, [gv, kv])
                    wa = plsc.load_gather(w1v, [kv, iv])
                    wb = plsc.load_gather(w1v, [kv, iv + LANES])
                    return (hh[0] + fgk * wa, hh[1] + fgk * wb)
                ha, hb = lax.fori_loop(0, D1, dot,
                                       (jnp.zeros((LANES,), _f32),
                                        jnp.zeros((LANES,), _f32)))
                ha = ha * (1.0 / (1.0 + jnp.exp(-ha)))
                hb = hb * (1.0 / (1.0 + jnp.exp(-hb)))
                mu = (jnp.sum(ha) + jnp.sum(hb)) * (1.0 / DH)
                var = (jnp.sum(ha * ha) + jnp.sum(hb * hb)) * (1.0 / DH) - mu * mu
                rstd = _rsqrt16(jnp.zeros((LANES,), _f32) + var + 1e-5)
                na = (ha - mu) * rstd * lgv[pl.ds(0, LANES)] \
                    + lbv[pl.ds(0, LANES)]
                nb_ = (hb - mu) * rstd * lgv[pl.ds(LANES, LANES)] \
                    + lbv[pl.ds(LANES, LANES)]
                w20a = w2v[pl.ds(0, LANES)]
                w20b = w2v[pl.ds(LANES, LANES)]
                w21a = w2v[pl.ds(2 * LANES, LANES)]
                w21b = w2v[pl.ds(3 * LANES, LANES)]
                o0 = jnp.sum(na * w20a) + jnp.sum(nb_ * w20b)
                o1 = jnp.sum(na * w21a) + jnp.sum(nb_ * w21b)
                t = 2 * g
                oa = jnp.where(iv == t, o0, ovs[0])
                oa = jnp.where(iv == t + 1, o1, oa)
                ob = jnp.where(iv == t - LANES, o0, ovs[1])
                ob = jnp.where(iv == t - LANES + 1, o1, ob)
                return (oa, ob)
            oa, ob = lax.fori_loop(0, LANES, graph,
                                   (jnp.zeros((LANES,), _f32),
                                    jnp.zeros((LANES,), _f32)))
            obuf[pl.ds(0, LANES)] = oa
            obuf[pl.ds(LANES, LANES)] = ob
            pltpu.sync_copy(obuf, out_h.at[pl.ds(w * 2 * LANES, 2 * LANES)])

    return k0a, k0c, k0b, klayer, k9, NPT, NPTP


def kernel(x, edge_index, batch, Wl0, Wr0, Wl_rest, Wr_rest,
           ro_W1, ln_g, ln_b, ro_W2):
    N, NIN = x.shape
    E = edge_index.shape[1]
    H = Wr0.shape[1]
    L = Wl_rest.shape[0] + 1
    k0a, k0c, k0b, klayer, k9, NPT, NPTP = _make_kernels(N, E, H, L)

    xp = jnp.pad(x.astype(_f32), ((0, 0), (0, H - NIN)))
    Wl0p = jnp.zeros((2 * H, H), _f32)
    Wl0p = Wl0p.at[0:NIN].set(Wl0[0:NIN]).at[H:H + NIN].set(Wl0[NIN:2 * NIN])
    Wr0p = jnp.zeros((H, H), _f32).at[0:NIN].set(Wr0)
    Wls = jnp.concatenate([Wl0p[None], Wl_rest], axis=0)
    Wrs = jnp.concatenate([Wr0p[None], Wr_rest], axis=0)
    Wls = jnp.pad(Wls.reshape(L, 2 * H * H), ((0, 0), (0, LANES)))
    Wrs = jnp.pad(Wrs.reshape(L, H * H), ((0, 0), (0, LANES)))

    D1 = 24 * (L + 1)
    DH = 4 * H
    W1p = jnp.zeros((D1, DH), _f32)
    W1p = W1p.at[0:NIN].set(ro_W1[0:NIN])
    W1p = W1p.at[H:H + NIN].set(ro_W1[NIN:2 * NIN])
    W1p = W1p.at[2 * H:2 * H + NIN].set(ro_W1[2 * NIN:3 * NIN])
    for s in range(1, L + 1):
        rb = 3 * NIN + (s - 1) * 3 * H
        W1p = W1p.at[24 * s:24 * s + 3 * H].set(ro_W1[rb:rb + 3 * H])
    W2t = ro_W2.T.reshape(-1)

    src = edge_index[0]
    dst = edge_index[1]
    batch2 = jnp.pad(batch.reshape(NW, NPT), ((0, 0), (0, NPTP - NPT)))

    eldu, elsu, cnt, rs = k0a(dst, src)
    elds, elss = k0c(eldu, elsu, cnt, rs)
    pools0, cntp, grs = k0b(xp, batch2)
    pools = [pools0]
    xc = xp
    for l in range(L):
        xc, pp = klayer(xc, elds, elss, cnt, rs, grs, Wls[l], Wrs[l])
        pools.append(pp)
    out = k9(cntp, W1p, ln_g, ln_b, W2t, *pools)
    return out.reshape(G, 2)


# speedup vs baseline: 1.1898x; 1.0934x over previous
import functools

import jax
import jax.numpy as jnp
from jax import lax
from jax.experimental import pallas as pl
from jax.experimental.pallas import tpu as pltpu
from jax.experimental.pallas import tpu_sc as plsc

NW = 32
LANES = 16
G = 256

_i32 = jnp.int32
_f32 = jnp.float32

_MESH = functools.partial(
    plsc.VectorSubcoreMesh, core_axis_name="c", subcore_axis_name="s",
    num_cores=2, num_subcores=16)

_CP = pltpu.CompilerParams(needs_layout_passes=False,
                           use_tc_tiling_on_sc=False)


def _wid():
    return lax.axis_index("s") * 2 + lax.axis_index("c")


def _iota():
    return lax.broadcasted_iota(_i32, (LANES,), 0)


def _splat(s):
    return jnp.zeros((LANES,), _i32) + s


def _rsqrt16(v):
    i = plsc.bitcast(v, _i32)
    y = plsc.bitcast(jnp.full((LANES,), 0x5F3759DF, _i32) - (i >> 1), _f32)
    for _ in range(3):
        y = y * (1.5 - 0.5 * v * y * y)
    return y


def _make_kernels(N, E, H, L):
    NPT = N // NW
    NPTP = -(-NPT // LANES) * LANES
    CH = 1024 if E >= NW * 2048 else max(16, min(1024, (E // NW) // 8 * 8))
    slack = max(1024, int(15 * (E // NW) ** 0.5))
    CAP = min(E + CH, E // NW + slack)
    CAP = min(-(-CAP // CH) * CH, -(-E // CH) * CH)
    CH0 = 16
    for c in range(4096, 15, -16):
        if E % c == 0:
            CH0 = c
            break
    H2 = 2 * H

    def _lanes():
        iv = _iota()
        m8 = iv < H
        hi8 = jnp.logical_not(m8)
        f8 = iv % H
        init_acc = jnp.where(m8, jnp.full((LANES,), -jnp.inf, _f32),
                             jnp.zeros((LANES,), _f32))
        return iv, m8, hi8, f8, init_acc

    @functools.partial(
        pl.kernel,
        out_type=(jax.ShapeDtypeStruct((NW, CAP), _i32),
                  jax.ShapeDtypeStruct((NW, CAP), _i32),
                  jax.ShapeDtypeStruct((NW, LANES), _i32),
                  jax.ShapeDtypeStruct((NW, NPTP), _i32)),
        mesh=_MESH(), compiler_params=_CP,
        scratch_types=[pltpu.VMEM((CH0,), _i32), pltpu.VMEM((CH0,), _i32),
                       pltpu.VMEM((CH0,), _i32), pltpu.VMEM((CH0,), _i32),
                       pltpu.VMEM((CAP,), _i32), pltpu.VMEM((CAP,), _i32),
                       pltpu.VMEM((NPTP,), _i32), pltpu.VMEM((NPTP,), _i32),
                       pltpu.VMEM((LANES,), _i32),
                       pltpu.SemaphoreType.DMA, pltpu.SemaphoreType.DMA,
                       pltpu.SemaphoreType.DMA, pltpu.SemaphoreType.DMA])
    def k0a(dst_h, src_h, eld_o, els_o, cnt_o, rs_o,
            dbuf0, sbuf0, dbuf1, sbuf1, eldv, elsv, degv, rsv, cbuf,
            smd0, sms0, smd1, sms1):
        iv, m8, hi8, f8, init_acc = _lanes()
        w = _wid()
        lo = w * NPT
        hi = lo + NPT
        z16 = jnp.zeros((LANES,), _i32)
        one16 = jnp.full((LANES,), 1, _i32)

        def init_e(i, _):
            eldv[pl.ds(i * LANES, LANES)] = z16
            elsv[pl.ds(i * LANES, LANES)] = z16
            return 0
        lax.fori_loop(0, CAP // LANES, init_e, 0)

        def init_d(i, _):
            degv[pl.ds(i * LANES, LANES)] = z16
            return 0
        lax.fori_loop(0, NPTP // LANES, init_d, 0)

        NCH0 = E // CH0
        bufs = ((dbuf0, sbuf0, smd0, sms0), (dbuf1, sbuf1, smd1, sms1))

        def start(c, slot):
            db, sb, sd, ss = bufs[slot]
            pltpu.async_copy(dst_h.at[pl.ds(c * CH0, CH0)], db, sd)
            pltpu.async_copy(src_h.at[pl.ds(c * CH0, CH0)], sb, ss)

        def scan_chunk(slot, cntv):
            db, sb, sd, ss = bufs[slot]
            pltpu.make_async_copy(dst_h.at[pl.ds(0, CH0)], db, sd).wait()
            pltpu.make_async_copy(src_h.at[pl.ds(0, CH0)], sb, ss).wait()

            def scan16(i, cv):
                d = db[pl.ds(i * LANES, LANES)]
                sv = sb[pl.ds(i * LANES, LANES)]
                m = jnp.logical_and(d >= lo, d < hi)
                mi = m.astype(_i32)
                cums = jnp.cumsum(mi)
                pos = cv + cums - mi
                dl = d - lo
                dlc = jnp.clip(dl, 0, NPT - 1)
                plsc.store_scatter(eldv, [pos], dl, mask=m)
                plsc.store_scatter(elsv, [pos], sv, mask=m)
                plsc.addupdate_scatter(degv, [dlc], one16, mask=m)
                return cv + plsc.all_reduce_population_count(m)
            return lax.fori_loop(0, CH0 // LANES, scan16, cntv)

        start(0, 0)

        def body2(c2, cntv):
            c_even = 2 * c2

            @pl.when(c_even + 1 < NCH0)
            def _():
                start(c_even + 1, 1)
            cntv2 = scan_chunk(0, cntv)

            @pl.when(c_even + 2 < NCH0)
            def _():
                start(c_even + 2, 0)

            def odd(cv):
                return scan_chunk(1, cv)
            cntv3 = lax.cond(c_even + 1 < NCH0, odd, lambda cv: cv, cntv2)
            return cntv3
        cntv = lax.fori_loop(0, (NCH0 + 1) // 2, body2,
                             jnp.zeros((LANES,), _i32))

        def prefix(i, carry):
            sl = pl.ds(i * LANES, LANES)
            d16 = degv[sl]
            c = jnp.cumsum(d16)
            rsv[sl] = carry + c - d16
            return carry + c[LANES - 1]
        lax.fori_loop(0, NPTP // LANES, prefix, jnp.int32(0))

        cbuf[...] = cntv
        pltpu.sync_copy(eldv, eld_o.at[w])
        pltpu.sync_copy(elsv, els_o.at[w])
        pltpu.sync_copy(cbuf, cnt_o.at[w])
        pltpu.sync_copy(rsv, rs_o.at[w])

    @functools.partial(
        pl.kernel,
        out_type=(jax.ShapeDtypeStruct((NW, CAP), _i32),
                  jax.ShapeDtypeStruct((NW, CAP), _i32)),
        mesh=_MESH(), compiler_params=_CP,
        scratch_types=[pltpu.VMEM((CH,), _i32), pltpu.VMEM((CH,), _i32),
                       pltpu.VMEM((CAP,), _i32), pltpu.VMEM((CAP,), _i32),
                       pltpu.VMEM((NPTP,), _i32), pltpu.VMEM((LANES,), _i32)])
    def k0c(eld_h, els_h, cnt_h, rs_h, elds_o, elss_o,
            dbuf, sbuf, eldv, elsv, wpv, cbuf):
        w = _wid()
        pltpu.sync_copy(cnt_h.at[w], cbuf)
        pltpu.sync_copy(rs_h.at[w], wpv)
        n_w = cbuf[...][0]
        z16 = jnp.zeros((LANES,), _i32)

        def init_e(i, _):
            eldv[pl.ds(i * LANES, LANES)] = z16
            elsv[pl.ds(i * LANES, LANES)] = z16
            return 0
        lax.fori_loop(0, CAP // LANES, init_e, 0)

        def chunk(c, _):
            c0 = c * CH
            cl = jnp.minimum(CH, n_w - c0)
            pltpu.sync_copy(eld_h.at[w, pl.ds(c0, CH)], dbuf)
            pltpu.sync_copy(els_h.at[w, pl.ds(c0, CH)], sbuf)

            def place(e, _):
                ev = _splat(e)
                dv = plsc.load_gather(dbuf, [ev])
                sv = plsc.load_gather(sbuf, [ev])
                pv = plsc.load_gather(wpv, [dv])
                plsc.store_scatter(wpv, [dv], pv + 1)
                plsc.store_scatter(eldv, [pv], dv)
                plsc.store_scatter(elsv, [pv], sv)
                return 0
            lax.fori_loop(0, cl, place, 0)
            return 0
        lax.fori_loop(0, (n_w + CH - 1) // CH, chunk, 0)
        pltpu.sync_copy(eldv, elds_o.at[w])
        pltpu.sync_copy(elsv, elss_o.at[w])

    @functools.partial(
        pl.kernel,
        out_type=(jax.ShapeDtypeStruct((NW, G, LANES), _f32),
                  jax.ShapeDtypeStruct((NW, G), _f32),
                  jax.ShapeDtypeStruct((NW, G), _i32)),
        mesh=_MESH(), compiler_params=_CP,
        scratch_types=[pltpu.VMEM((NPTP, H), _f32), pltpu.VMEM((NPTP,), _i32),
                       pltpu.VMEM((G, LANES), _f32), pltpu.VMEM((G,), _f32),
                       pltpu.VMEM((G,), _i32)])
    def k0b(x_h, batch_h, pool_o, cnt_o, grs_o, xv, bv, pacc, cacc, grsv):
        iv, m8, hi8, f8, init_acc = _lanes()
        w = _wid()
        lo = w * NPT
        pltpu.sync_copy(x_h.at[pl.ds(lo, NPT)], xv.at[pl.ds(0, NPT)])
        pltpu.sync_copy(batch_h.at[w], bv)

        def initp(g, _):
            plsc.store_scatter(pacc, [_splat(g), iv], init_acc)
            return 0
        lax.fori_loop(0, G, initp, 0)

        def initc(i, _):
            cacc[pl.ds(i * LANES, LANES)] = jnp.zeros((LANES,), _f32)
            return 0
        lax.fori_loop(0, G // LANES, initc, 0)

        def node(n, _):
            nv = _splat(n)
            gv = plsc.load_gather(bv, [nv])
            y = plsc.load_gather(xv, [nv, f8])
            p = plsc.load_gather(pacc, [gv, iv])
            plsc.store_scatter(pacc, [gv, iv],
                               jnp.where(m8, jnp.maximum(p, y), p + y))
            c = plsc.load_gather(cacc, [gv])
            plsc.store_scatter(cacc, [gv], c + 1.0)
            return 0
        lax.fori_loop(0, NPT, node, 0)

        def gprefix(i, carry):
            sl = pl.ds(i * LANES, LANES)
            d16 = cacc[sl].astype(_i32)
            c = jnp.cumsum(d16)
            grsv[sl] = carry + c - d16
            return carry + c[LANES - 1]
        lax.fori_loop(0, G // LANES, gprefix, jnp.int32(0))
        pltpu.sync_copy(pacc, pool_o.at[w])
        pltpu.sync_copy(cacc, cnt_o.at[w])
        pltpu.sync_copy(grsv, grs_o.at[w])

    @functools.partial(
        pl.kernel,
        out_type=(jax.ShapeDtypeStruct((N, H), _f32),
                  jax.ShapeDtypeStruct((NW, G, LANES), _f32)),
        mesh=_MESH(), compiler_params=_CP,
        scratch_types=[pltpu.VMEM((CH,), _i32), pltpu.VMEM((CH,), _i32),
                       pltpu.VMEM((CH, H), _f32),
                       pltpu.VMEM((NPTP * H,), _f32),
                       pltpu.VMEM((NPTP * H,), _f32),
                       pltpu.VMEM((NPTP, H), _f32),
                       pltpu.VMEM((NPTP,), _f32), pltpu.VMEM((NPTP,), _i32),
                       pltpu.VMEM((G,), _i32),
                       pltpu.VMEM((G, LANES), _f32),
                       pltpu.VMEM((H2 * H + LANES,), _f32),
                       pltpu.VMEM((H * H + LANES,), _f32),
                       pltpu.VMEM((LANES,), _i32)])
    def klayer(x_h, eld_h, els_h, cnt_h, rs_h, grs_h, wl_h, wr_h,
               xn_h, pool_o,
               dbuf, sbuf, msgv, mxv, smv, xbuf, rdv, rsv, bv, pacc,
               wlv, wrv, cntv_b):
        iv, m8, hi8, f8, init_acc = _lanes()
        w = _wid()
        lo = w * NPT
        pltpu.sync_copy(cnt_h.at[w], cntv_b)
        pltpu.sync_copy(wl_h, wlv)
        pltpu.sync_copy(wr_h, wrv)
        pltpu.sync_copy(x_h.at[pl.ds(lo, NPT)], xbuf.at[pl.ds(0, NPT)])
        pltpu.sync_copy(rs_h.at[w], rsv)
        pltpu.sync_copy(grs_h.at[w], bv)
        n_w = cntv_b[...][0]
        wls = []
        for k in range(H2):
            row = wlv[pl.ds(k * H, LANES)]
            wls.append([row[j] for j in range(H)])
        wrs = []
        for i in range(H):
            row = wrv[pl.ds(i * H, LANES)]
            wrs.append([row[j] for j in range(H)])

        neg16 = jnp.full((LANES,), -jnp.inf, _f32)
        z16f = jnp.zeros((LANES,), _f32)
        rows2 = (iv >= H).astype(_i32)

        def inita(i, _):
            mxv[pl.ds(i * LANES, LANES)] = neg16
            smv[pl.ds(i * LANES, LANES)] = z16f
            return 0
        lax.fori_loop(0, NPTP * H // LANES, inita, 0)

        def rdeg(i, _):
            sl = pl.ds(i * LANES, LANES)
            a = rsv[sl]
            b = plsc.load_gather(rsv, [jnp.minimum(_splat(i * LANES) + iv + 1,
                                                   _splat(NPTP - 1))])
            bfix = jnp.where(_splat(i * LANES) + iv + 1 >= NPTP,
                             _splat(n_w), b)
            deg = (bfix - a).astype(_f32)
            rdv[sl] = 1.0 / jnp.maximum(deg, 1.0)
            return 0
        lax.fori_loop(0, NPTP // LANES, rdeg, 0)

        def chunk(c, _):
            c0 = c * CH
            cl = jnp.minimum(CH, n_w - c0)
            cend = c0 + cl
            pltpu.sync_copy(eld_h.at[w, pl.ds(c0, CH)], dbuf)
            pltpu.sync_copy(els_h.at[w, pl.ds(c0, CH)], sbuf)
            pltpu.sync_copy(x_h.at[sbuf], msgv)
            nlo = plsc.load_gather(dbuf, [_splat(0)])[0]
            nhi = plsc.load_gather(dbuf, [_splat(cl - 1)])[0]
            m0 = nlo // 2

            def pairs(mi_, _):
                m = m0 + mi_
                n0 = 2 * m
                g3 = plsc.load_gather(rsv, [_splat(n0) + jnp.minimum(iv, 2)])
                a0 = g3[0]
                a1 = g3[1]
                a2 = g3[2]
                kA0 = jnp.maximum(a0, c0)
                dA = jnp.maximum(jnp.minimum(a1, cend) - kA0, 0)
                kB0 = jnp.maximum(a1, c0)
                dB = jnp.maximum(jnp.minimum(a2, cend) - kB0, 0)
                kmax = jnp.maximum(dA, dB)
                base = jnp.where(m8, _splat(kA0), _splat(kB0)) - c0
                dcnt = jnp.where(m8, _splat(dA), _splat(dB))

                def acc_k(k, carry):
                    am, asum = carry
                    rowl = base + k
                    valid = (jnp.zeros((LANES,), _i32) + k) < dcnt
                    rc = jnp.minimum(rowl, CH - 1)
                    msg = plsc.load_gather(msgv, [rc, f8])
                    am = jnp.maximum(am, jnp.where(valid, msg, -jnp.inf))
                    asum = asum + jnp.where(valid, msg, 0.0)
                    return (am, asum)
                am, asum = lax.fori_loop(0, kmax, acc_k, (neg16, z16f))
                off = m * LANES
                sl = pl.ds(off, LANES)
                mxv[sl] = jnp.maximum(mxv[sl], am)
                smv[sl] = smv[sl] + asum
                return 0
            lax.fori_loop(0, nhi // 2 - m0 + 1, pairs, 0)
            return 0
        lax.fori_loop(0, (n_w + CH - 1) // CH, chunk, 0)

        def fin(i, _):
            sl = pl.ds(i * LANES, LANES)
            v = mxv[sl]
            mxv[sl] = jnp.where(v == -jnp.inf, 0.0, v)
            r2 = _splat(2 * i) + rows2
            rd = plsc.load_gather(rdv, [r2])
            smv[sl] = smv[sl] * rd
            return 0
        lax.fori_loop(0, NPTP * H // LANES, fin, 0)

        def blk(nb, _):
            nodes = _splat(nb * LANES) + iv
            nflat = nodes * H
            h = [jnp.zeros((LANES,), _f32) for _ in range(H)]
            for k in range(H):
                ak = plsc.load_gather(mxv, [nflat + k])
                for j in range(H):
                    h[j] = h[j] + ak * wls[k][j]
            for k in range(H):
                ak = plsc.load_gather(smv, [nflat + k])
                for j in range(H):
                    h[j] = h[j] + ak * wls[H + k][j]
            for i in range(H):
                xi = plsc.load_gather(xbuf, [nodes, _splat(i)])
                for j in range(H):
                    h[j] = h[j] + xi * wrs[i][j]
            for j in range(H):
                y = h[j] * (1.0 / (1.0 + jnp.exp(-h[j])))
                plsc.store_scatter(xbuf, [nodes, _splat(j)], y)
            return 0
        lax.fori_loop(0, NPTP // LANES, blk, 0)

        def gpool(g, _):
            gg = plsc.load_gather(bv, [jnp.minimum(_splat(g) + iv,
                                                   _splat(G - 1))])
            a = gg[0]
            b = jnp.where(g + 1 >= G, jnp.int32(NPT),
                          jnp.minimum(gg[1], NPT))
            bs = _splat(b)

            def acc2(k, carry):
                am, asum = carry
                r = _splat(a + 2 * k) + rows2
                rsw = _splat(a + 2 * k + 1) - rows2
                v1 = r < bs
                v2 = rsw < bs
                y = plsc.load_gather(
                    xbuf, [jnp.minimum(r, _splat(NPTP - 1)), f8])
                ysw = plsc.load_gather(
                    xbuf, [jnp.minimum(rsw, _splat(NPTP - 1)), f8])
                am = jnp.maximum(am, jnp.maximum(
                    jnp.where(v1, y, -jnp.inf), jnp.where(v2, ysw, -jnp.inf)))
                asum = asum + jnp.where(v1, y, 0.0) + jnp.where(v2, ysw, 0.0)
                return (am, asum)
            am, asum = lax.fori_loop(0, (b - a + 1) // 2, acc2,
                                     (neg16, z16f))
            plsc.store_scatter(pacc, [_splat(g), iv],
                               jnp.where(m8, am, asum))
            return 0
        lax.fori_loop(0, G, gpool, 0)

        pltpu.sync_copy(xbuf.at[pl.ds(0, NPT)], xn_h.at[pl.ds(lo, NPT)])
        pltpu.sync_copy(pacc, pool_o.at[w])

    NST = L + 1
    D1 = 24 * NST
    FW = -(-D1 // LANES) * LANES + 8
    DH = 4 * H
    NT9 = G // LANES

    @functools.partial(
        pl.kernel,
        out_type=jax.ShapeDtypeStruct((G * 2,), _f32),
        mesh=_MESH(), compiler_params=_CP,
        scratch_types=[pltpu.VMEM((LANES, LANES), _f32),
                       pltpu.VMEM((LANES,), _f32),
                       pltpu.VMEM((LANES, FW), _f32),
                       pltpu.VMEM((D1, DH), _f32),
                       pltpu.VMEM((DH,), _f32), pltpu.VMEM((DH,), _f32),
                       pltpu.VMEM((2 * DH,), _f32),
                       pltpu.VMEM((2 * LANES,), _f32)])
    def k9(cnt_h, w1_h, lng_h, lnb_h, w2_h, *rest):
        pools_h = rest[:NST]
        out_h = rest[NST]
        pbuf, rcv, fbuf, w1v, lgv, lbv, w2v, obuf = rest[NST + 1:]
        iv, m8, hi8, f8, init_acc = _lanes()
        w = _wid()

        @pl.when(w < NT9)
        def _():
            g0 = w * LANES
            pltpu.sync_copy(w1_h, w1v)
            pltpu.sync_copy(lng_h, lgv)
            pltpu.sync_copy(lnb_h, lbv)
            pltpu.sync_copy(w2_h, w2v)

            def csum(p, cv):
                pltpu.sync_copy(cnt_h.at[p, pl.ds(g0, LANES)], rcv)
                return cv + rcv[...]
            cntv = lax.fori_loop(0, NW, csum, jnp.zeros((LANES,), _f32))
            rcv[...] = 1.0 / jnp.maximum(cntv, 1.0)

            for st in range(NST):
                def comb(p, accs, st=st):
                    pltpu.sync_copy(pools_h[st].at[p, pl.ds(g0, LANES)], pbuf)
                    out = []
                    for q in range(LANES):
                        v = plsc.load_gather(pbuf, [_splat(q), iv])
                        out.append(jnp.where(m8, jnp.maximum(accs[q], v),
                                             accs[q] + v))
                    return tuple(out)
                accs = lax.fori_loop(0, NW, comb,
                                     tuple(init_acc for _ in range(LANES)))
                for q in range(LANES):
                    vq = jnp.where(accs[q] == -jnp.inf, 0.0, accs[q])
                    rcq = plsc.load_gather(rcv, [_splat(q)])
                    meanv = vq * rcq
                    qv = _splat(q)
                    cmean = jnp.maximum(st * 24 + iv - H, 0)
                    plsc.store_scatter(fbuf, [qv, cmean], meanv, mask=hi8)
                    plsc.store_scatter(fbuf, [qv, st * 24 + H + iv], vq,
                                       mask=m8)
                    plsc.store_scatter(fbuf, [qv, st * 24 + H + iv], vq,
                                       mask=hi8)

            def graph(g, ovs):
                gv = _splat(g)

                def dot(k, hh):
                    kv = _splat(k)
                    fgk = plsc.load_gather(fbuf, [gv, kv])
                    wa = plsc.load_gather(w1v, [kv, iv])
                    wb = plsc.load_gather(w1v, [kv, iv + LANES])
                    return (hh[0] + fgk * wa, hh[1] + fgk * wb)
                ha, hb = lax.fori_loop(0, D1, dot,
                                       (jnp.zeros((LANES,), _f32),
                                        jnp.zeros((LANES,), _f32)))
                ha = ha * (1.0 / (1.0 + jnp.exp(-ha)))
                hb = hb * (1.0 / (1.0 + jnp.exp(-hb)))
                mu = (jnp.sum(ha) + jnp.sum(hb)) * (1.0 / DH)
                var = (jnp.sum(ha * ha) + jnp.sum(hb * hb)) * (1.0 / DH) - mu * mu
                rstd = _rsqrt16(jnp.zeros((LANES,), _f32) + var + 1e-5)
                na = (ha - mu) * rstd * lgv[pl.ds(0, LANES)] \
                    + lbv[pl.ds(0, LANES)]
                nb_ = (hb - mu) * rstd * lgv[pl.ds(LANES, LANES)] \
                    + lbv[pl.ds(LANES, LANES)]
                w20a = w2v[pl.ds(0, LANES)]
                w20b = w2v[pl.ds(LANES, LANES)]
                w21a = w2v[pl.ds(2 * LANES, LANES)]
                w21b = w2v[pl.ds(3 * LANES, LANES)]
                o0 = jnp.sum(na * w20a) + jnp.sum(nb_ * w20b)
                o1 = jnp.sum(na * w21a) + jnp.sum(nb_ * w21b)
                t = 2 * g
                oa = jnp.where(iv == t, o0, ovs[0])
                oa = jnp.where(iv == t + 1, o1, oa)
                ob = jnp.where(iv == t - LANES, o0, ovs[1])
                ob = jnp.where(iv == t - LANES + 1, o1, ob)
                return (oa, ob)
            oa, ob = lax.fori_loop(0, LANES, graph,
                                   (jnp.zeros((LANES,), _f32),
                                    jnp.zeros((LANES,), _f32)))
            obuf[pl.ds(0, LANES)] = oa
            obuf[pl.ds(LANES, LANES)] = ob
            pltpu.sync_copy(obuf, out_h.at[pl.ds(w * 2 * LANES, 2 * LANES)])

    return k0a, k0c, k0b, klayer, k9, NPT, NPTP


def kernel(x, edge_index, batch, Wl0, Wr0, Wl_rest, Wr_rest,
           ro_W1, ln_g, ln_b, ro_W2):
    N, NIN = x.shape
    E = edge_index.shape[1]
    H = Wr0.shape[1]
    L = Wl_rest.shape[0] + 1
    k0a, k0c, k0b, klayer, k9, NPT, NPTP = _make_kernels(N, E, H, L)

    xp = jnp.pad(x.astype(_f32), ((0, 0), (0, H - NIN)))
    Wl0p = jnp.zeros((2 * H, H), _f32)
    Wl0p = Wl0p.at[0:NIN].set(Wl0[0:NIN]).at[H:H + NIN].set(Wl0[NIN:2 * NIN])
    Wr0p = jnp.zeros((H, H), _f32).at[0:NIN].set(Wr0)
    Wls = jnp.concatenate([Wl0p[None], Wl_rest], axis=0)
    Wrs = jnp.concatenate([Wr0p[None], Wr_rest], axis=0)
    Wls = jnp.pad(Wls.reshape(L, 2 * H * H), ((0, 0), (0, LANES)))
    Wrs = jnp.pad(Wrs.reshape(L, H * H), ((0, 0), (0, LANES)))

    D1 = 24 * (L + 1)
    DH = 4 * H
    W1p = jnp.zeros((D1, DH), _f32)
    W1p = W1p.at[0:NIN].set(ro_W1[0:NIN])
    W1p = W1p.at[H:H + NIN].set(ro_W1[NIN:2 * NIN])
    W1p = W1p.at[2 * H:2 * H + NIN].set(ro_W1[2 * NIN:3 * NIN])
    for s in range(1, L + 1):
        rb = 3 * NIN + (s - 1) * 3 * H
        W1p = W1p.at[24 * s:24 * s + 3 * H].set(ro_W1[rb:rb + 3 * H])
    W2t = ro_W2.T.reshape(-1)

    src = edge_index[0]
    dst = edge_index[1]
    batch2 = jnp.pad(batch.reshape(NW, NPT), ((0, 0), (0, NPTP - NPT)))

    eldu, elsu, cnt, rs = k0a(dst, src)
    elds, elss = k0c(eldu, elsu, cnt, rs)
    pools0, cntp, grs = k0b(xp, batch2)
    pools = [pools0]
    xc = xp
    for l in range(L):
        xc, pp = klayer(xc, elds, elss, cnt, rs, grs, Wls[l], Wrs[l])
        pools.append(pp)
    out = k9(cntp, W1p, ln_g, ln_b, W2t, *pools)
    return out.reshape(G, 2)
